# Initial kernel scaffold; baseline (speedup 1.0000x reference)
import jax
import jax.numpy as jnp
from jax.experimental import pallas as pl


def kernel(segment_ids, table):
    raise NotImplementedError("write your pallas kernel here")


# speedup vs baseline: 5.8978x; 5.8978x over previous
import jax
import jax.numpy as jnp
from jax.experimental import pallas as pl

D_MODEL = 768
TOK_BLOCK = 2048


def _embed_body(seg_ref, tab_ref, out_ref):
    seg = seg_ref[0, 0, :]
    t0 = tab_ref[0, :]
    t1 = tab_ref[1, :]
    out_ref[...] = jnp.where((seg[:, None] == 0), t0[None, :], t1[None, :])


def kernel(segment_ids, table):
    b, s = segment_ids.shape
    n_tok = b * s
    nb = n_tok // TOK_BLOCK
    seg3 = segment_ids.reshape(nb, 1, TOK_BLOCK).astype(jnp.int32)
    out = pl.pallas_call(
        _embed_body,
        grid=(nb,),
        in_specs=[
            pl.BlockSpec((1, 1, TOK_BLOCK), lambda i: (i, 0, 0)),
            pl.BlockSpec((2, D_MODEL), lambda i: (0, 0)),
        ],
        out_specs=pl.BlockSpec((TOK_BLOCK, D_MODEL), lambda i: (i, 0)),
        out_shape=jax.ShapeDtypeStruct((n_tok, D_MODEL), table.dtype),
    )(seg3, table)
    return out.reshape(b, s, D_MODEL)
